# baseline (speedup 1.0000x reference)
import functools

import jax
import jax.numpy as jnp
from jax import lax
from jax.experimental import pallas as pl
from jax.experimental.pallas import tpu as pltpu
from jax.experimental.pallas import tpu_sc as plsc

V = 1000000
D = 64
R = 16
N = 1024 * 200

NC = 2
NS = 16
NW = NC * NS
B_PER_W = N // NW
CHUNK = 640
NCHUNKS = B_PER_W // CHUNK
ACHUNK = 640
ANCHUNKS = B_PER_W // ACHUNK


def _worker_id():
    return lax.axis_index("s") * NC + lax.axis_index("c")


@functools.cache
def _sc_kernels():
    mesh = plsc.VectorSubcoreMesh(core_axis_name="c", subcore_axis_name="s")

    @functools.partial(
        pl.kernel,
        out_type=jax.ShapeDtypeStruct((2 * N, 32), jnp.float32),
        mesh=mesh,
        compiler_params=pltpu.CompilerParams(use_tc_tiling_on_sc=False),
        scratch_types=[
            pltpu.VMEM((2 * B_PER_W,), jnp.int32),
            pltpu.VMEM((2 * CHUNK, 32), jnp.float32),
            pltpu.VMEM((2 * CHUNK, 32), jnp.float32),
            pltpu.SemaphoreType.DMA,
            pltpu.SemaphoreType.DMA,
        ],
    )
    def base_gather(idx2_hbm, table_hbm, out_hbm, idx_v, b0, b1, sem0, sem1):
        base = _worker_id() * 2 * B_PER_W
        pltpu.sync_copy(idx2_hbm.at[pl.ds(base, 2 * B_PER_W)], idx_v)
        bufs = (b0, b1)
        sems = (sem0, sem1)
        cps = [None, None]

        def start(k):
            j = k % 2
            cps[j] = pltpu.async_copy(
                table_hbm.at[idx_v.at[pl.ds(2 * k * CHUNK, 2 * CHUNK)]],
                bufs[j], sems[j])

        start(0)
        for k in range(NCHUNKS):
            if k + 1 < NCHUNKS:
                start(k + 1)
            j = k % 2
            cps[j].wait()
            pltpu.sync_copy(
                bufs[j], out_hbm.at[pl.ds(base + 2 * k * CHUNK, 2 * CHUNK)])

    @functools.partial(
        pl.kernel,
        out_type=jax.ShapeDtypeStruct((R * V,), jnp.float32),
        mesh=mesh,
        compiler_params=pltpu.CompilerParams(use_tc_tiling_on_sc=False),
        scratch_types=[],
    )
    def flatten_a(a_hbm, out_hbm):
        wid = _worker_id()
        r = wid // 2
        c0 = (wid % 2) * (V // 2)
        pltpu.sync_copy(
            a_hbm.at[r, pl.ds(c0, V // 2)],
            out_hbm.at[pl.ds(r * V + c0, V // 2)],
        )

    @functools.partial(
        pl.kernel,
        out_type=jax.ShapeDtypeStruct((N * R,), jnp.float32),
        mesh=mesh,
        compiler_params=pltpu.CompilerParams(use_tc_tiling_on_sc=False),
        scratch_types=[
            pltpu.VMEM((B_PER_W * R,), jnp.int32),
            pltpu.VMEM((ACHUNK * R,), jnp.float32),
            pltpu.VMEM((ACHUNK * R,), jnp.float32),
            pltpu.SemaphoreType.DMA,
            pltpu.SemaphoreType.DMA,
        ],
    )
    def lora_gather(idx16_hbm, aflat_hbm, out_hbm, idx_v, b0, b1, sem0, sem1):
        base = _worker_id() * B_PER_W * R
        pltpu.sync_copy(idx16_hbm.at[pl.ds(base, B_PER_W * R)], idx_v)
        bufs = (b0, b1)
        sems = (sem0, sem1)
        cps = [None, None]
        n_el = ACHUNK * R

        def start(k):
            j = k % 2
            cps[j] = pltpu.async_copy(
                aflat_hbm.at[idx_v.at[pl.ds(k * n_el, n_el)]], bufs[j], sems[j])

        start(0)
        for k in range(ANCHUNKS):
            if k + 1 < ANCHUNKS:
                start(k + 1)
            j = k % 2
            cps[j].wait()
            pltpu.sync_copy(bufs[j], out_hbm.at[pl.ds(base + k * n_el, n_el)])

    return base_gather, lora_gather


_BN2 = 512
_BN8 = _BN2 // 4


def _fuse_body(ar8_ref, base2_ref, b8_ref, out_ref):
    lora8 = jnp.dot(ar8_ref[...], b8_ref[...],
                    preferred_element_type=jnp.float32)
    lora2 = lora8.reshape(_BN2, 128)
    out_ref[...] = base2_ref[...] + lora2


_fuse = pl.pallas_call(
    _fuse_body,
    grid=(N // (2 * _BN2),),
    in_specs=[
        pl.BlockSpec((_BN8, 128), lambda i: (i, 0)),
        pl.BlockSpec((_BN2, 128), lambda i: (i, 0)),
        pl.BlockSpec((128, 512), lambda i: (0, 0)),
    ],
    out_specs=pl.BlockSpec((_BN2, 128), lambda i: (i, 0)),
    out_shape=jax.ShapeDtypeStruct((N // 2, 128), jnp.float32),
)


def kernel(x, base_weight, lora_A, lora_B):
    Bsz, Ssz = x.shape
    idx = x.reshape(-1)
    idx2 = (2 * idx[:, None] + jnp.arange(2, dtype=jnp.int32)[None, :]).reshape(-1)
    idx16 = (idx[:, None] + (jnp.arange(R, dtype=jnp.int32) * V)[None, :]).reshape(-1)
    table2 = base_weight.reshape(2 * V, 32)
    aflat = lora_A.reshape(R * V)
    b8 = jnp.kron(jnp.eye(8, dtype=jnp.float32), lora_B.T)
    base_gather, lora_gather = _sc_kernels()
    rows = base_gather(idx2, table2)
    ar = lora_gather(idx16, aflat)
    out2 = _fuse(ar.reshape(N // 8, 128), rows.reshape(N // 2, 128), b8)
    return out2.reshape(Bsz, Ssz, D)


# speedup vs baseline: 1.6012x; 1.6012x over previous
import functools

import jax
import jax.numpy as jnp
from jax import lax
from jax.experimental import pallas as pl
from jax.experimental.pallas import tpu as pltpu
from jax.experimental.pallas import tpu_sc as plsc

V = 1000000
D = 64
R = 16
N = 1024 * 200

NC = 2
NS = 16
NW = NC * NS
B_PER_W = N // NW
CHUNK = 640
NCHUNKS = B_PER_W // CHUNK


def _worker_id():
    return lax.axis_index("s") * NC + lax.axis_index("c")


@functools.cache
def _sc_kernels():
    mesh = plsc.VectorSubcoreMesh(core_axis_name="c", subcore_axis_name="s")

    @functools.partial(
        pl.kernel,
        out_type=jax.ShapeDtypeStruct((N, D), jnp.float32),
        mesh=mesh,
        compiler_params=pltpu.CompilerParams(use_tc_tiling_on_sc=False),
        scratch_types=[
            pltpu.VMEM((B_PER_W,), jnp.int32),
            pltpu.VMEM((CHUNK, D), jnp.float32),
            pltpu.VMEM((CHUNK, D), jnp.float32),
            pltpu.SemaphoreType.DMA,
            pltpu.SemaphoreType.DMA,
        ],
    )
    def base_gather(idx_hbm, table_hbm, out_hbm, idx_v, b0, b1, sem0, sem1):
        base = _worker_id() * B_PER_W
        pltpu.sync_copy(idx_hbm.at[pl.ds(base, B_PER_W)], idx_v)
        bufs = (b0, b1)
        sems = (sem0, sem1)
        cps = [None, None]

        def start(k):
            j = k % 2
            cps[j] = pltpu.async_copy(
                table_hbm.at[idx_v.at[pl.ds(k * CHUNK, CHUNK)]],
                bufs[j], sems[j])

        start(0)
        for k in range(NCHUNKS):
            if k + 1 < NCHUNKS:
                start(k + 1)
            j = k % 2
            cps[j].wait()
            pltpu.sync_copy(
                bufs[j], out_hbm.at[pl.ds(base + k * CHUNK, CHUNK)])

    @functools.partial(
        pl.kernel,
        out_type=jax.ShapeDtypeStruct((N, R), jnp.float32),
        mesh=mesh,
        compiler_params=pltpu.CompilerParams(use_tc_tiling_on_sc=False),
        scratch_types=[
            pltpu.VMEM((B_PER_W,), jnp.int32),
            pltpu.VMEM((B_PER_W, R), jnp.float32),
        ],
    )
    def at_gather(idx_hbm, at_hbm, out_hbm, idx_v, rows_v):
        base = _worker_id() * B_PER_W
        pltpu.sync_copy(idx_hbm.at[pl.ds(base, B_PER_W)], idx_v)
        pltpu.sync_copy(at_hbm.at[idx_v], rows_v)
        pltpu.sync_copy(rows_v, out_hbm.at[pl.ds(base, B_PER_W)])

    return base_gather, at_gather


_BT = 4096


def _tr_body(a_ref, at_ref):
    at_ref[...] = a_ref[...].T


_transpose = pl.pallas_call(
    _tr_body,
    grid=((V + _BT - 1) // _BT,),
    in_specs=[pl.BlockSpec((R, _BT), lambda i: (0, i))],
    out_specs=pl.BlockSpec((_BT, R), lambda i: (i, 0)),
    out_shape=jax.ShapeDtypeStruct((V, R), jnp.float32),
)


_BN = 2048


def _fuse_body(ar_ref, base_ref, bt_ref, out_ref):
    out_ref[...] = base_ref[...] + jnp.dot(
        ar_ref[...], bt_ref[...], preferred_element_type=jnp.float32)


_fuse = pl.pallas_call(
    _fuse_body,
    grid=(N // _BN,),
    in_specs=[
        pl.BlockSpec((_BN, R), lambda i: (i, 0)),
        pl.BlockSpec((_BN, D), lambda i: (i, 0)),
        pl.BlockSpec((R, D), lambda i: (0, 0)),
    ],
    out_specs=pl.BlockSpec((_BN, D), lambda i: (i, 0)),
    out_shape=jax.ShapeDtypeStruct((N, D), jnp.float32),
)


def kernel(x, base_weight, lora_A, lora_B):
    Bsz, Ssz = x.shape
    idx = x.reshape(-1)
    at = _transpose(lora_A)
    base_gather, at_gather = _sc_kernels()
    rows = base_gather(idx, base_weight)
    ar = at_gather(idx, at)
    out = _fuse(ar, rows, lora_B.T)
    return out.reshape(Bsz, Ssz, D)
